# baseline (speedup 1.0000x reference)
import functools

import jax
import jax.numpy as jnp
import numpy as np
from jax import lax
from jax.experimental import pallas as pl
from jax.experimental.pallas import tpu as pltpu
from jax.experimental.pallas import tpu_sc as plsc

DIM = 64
HALF = DIM // 2
MAX_POS = 8192
THETA = 10000.0
SEQ = 8192

_w = 1.0 / (THETA ** (np.arange(0, DIM, 2, dtype=np.float64) / DIM))
_hi_ang = np.outer(np.arange(MAX_POS // 128) * 128.0, _w)
_lo_ang = np.outer(np.arange(128).astype(np.float64), _w)
_THI = np.concatenate([np.cos(_hi_ang), np.sin(_hi_ang)], axis=1).astype(np.float32)
_TLO = np.concatenate([np.cos(_lo_ang), np.sin(_lo_ang)], axis=1).astype(np.float32)

_NC, _NS = 2, 16
_NW = _NC * _NS
_CHUNK = SEQ // _NW
_G = _CHUNK // 16


@functools.partial(
    pl.kernel,
    mesh=plsc.VectorSubcoreMesh(core_axis_name="c", subcore_axis_name="s"),
    out_type=(
        jax.ShapeDtypeStruct((SEQ, DIM), jnp.float32),
        jax.ShapeDtypeStruct((SEQ, DIM), jnp.float32),
    ),
    scratch_types=[
        pltpu.VMEM((_CHUNK,), jnp.int32),
        pltpu.VMEM((MAX_POS // 128, DIM), jnp.float32),
        pltpu.VMEM((128, DIM), jnp.float32),
        pltpu.VMEM((_CHUNK, DIM), jnp.float32),
        pltpu.VMEM((_CHUNK, DIM), jnp.float32),
        pltpu.SemaphoreType.DMA,
        pltpu.SemaphoreType.DMA,
        pltpu.SemaphoreType.DMA,
    ],
    compiler_params=pltpu.CompilerParams(
        use_tc_tiling_on_sc=False, needs_layout_passes=False
    ),
)
def _rope_trig(thi_hbm, tlo_hbm, idx_hbm, cos_out, sin_out,
               idx_v, thi_v, tlo_v, cos_v, sin_v, sem_c, sem_s, sem_t):
    wid = lax.axis_index("s") * _NC + lax.axis_index("c")
    base = wid * _CHUNK
    cp1 = pltpu.async_copy(idx_hbm.at[pl.ds(base, _CHUNK)], idx_v, sem_c)
    cp2 = pltpu.async_copy(thi_hbm, thi_v, sem_s)
    cp3 = pltpu.async_copy(tlo_hbm, tlo_v, sem_t)
    cp1.wait()
    cp2.wait()
    cp3.wait()

    lanes = lax.iota(jnp.int32, 16)

    def _group(g):
        idxv = idx_v[pl.ds(g * 16, 16)]
        for j in range(16):
            p = g * 16 + j
            bvec = jnp.take_along_axis(
                idxv, jnp.full((16,), j, jnp.int32), axis=0,
                mode="promise_in_bounds",
            )
            hi = lax.shift_right_logical(bvec, 7)
            lo = lax.bitwise_and(bvec, 127)
            ch0 = plsc.load_gather(thi_v, [hi, lanes])
            ch1 = plsc.load_gather(thi_v, [hi, 16 + lanes])
            sh0 = plsc.load_gather(thi_v, [hi, 32 + lanes])
            sh1 = plsc.load_gather(thi_v, [hi, 48 + lanes])
            cl0 = plsc.load_gather(tlo_v, [lo, lanes])
            cl1 = plsc.load_gather(tlo_v, [lo, 16 + lanes])
            sl0 = plsc.load_gather(tlo_v, [lo, 32 + lanes])
            sl1 = plsc.load_gather(tlo_v, [lo, 48 + lanes])
            c0 = ch0 * cl0 - sh0 * sl0
            c1 = ch1 * cl1 - sh1 * sl1
            s0 = sh0 * cl0 + ch0 * sl0
            s1 = sh1 * cl1 + ch1 * sl1
            cos_v[p, pl.ds(0, 16)] = c0
            cos_v[p, pl.ds(16, 16)] = c1
            cos_v[p, pl.ds(32, 16)] = c0
            cos_v[p, pl.ds(48, 16)] = c1
            sin_v[p, pl.ds(0, 16)] = s0
            sin_v[p, pl.ds(16, 16)] = s1
            sin_v[p, pl.ds(32, 16)] = s0
            sin_v[p, pl.ds(48, 16)] = s1

    stores = []
    gq = _G // 4
    rq = _CHUNK // 4
    for h in range(4):
        plsc.parallel_loop(h * gq, (h + 1) * gq, 1, unroll=2)(_group)
        lo_r = h * rq
        stores.append(pltpu.async_copy(
            cos_v.at[pl.ds(lo_r, rq)],
            cos_out.at[pl.ds(base + lo_r, rq)], sem_c))
        stores.append(pltpu.async_copy(
            sin_v.at[pl.ds(lo_r, rq)],
            sin_out.at[pl.ds(base + lo_r, rq)], sem_s))
    for cp in stores:
        cp.wait()


def kernel(x, position_ids):
    thi = jnp.asarray(_THI)
    tlo = jnp.asarray(_TLO)
    idx = position_ids.reshape(SEQ).astype(jnp.int32)
    cos, sin = _rope_trig(thi, tlo, idx)
    cos = cos.reshape(1, 1, SEQ, DIM).astype(x.dtype)
    sin = sin.reshape(1, 1, SEQ, DIM).astype(x.dtype)
    return cos, sin


# speedup vs baseline: 1.0980x; 1.0980x over previous
import functools

import jax
import jax.numpy as jnp
import numpy as np
from jax import lax
from jax.experimental import pallas as pl
from jax.experimental.pallas import tpu as pltpu
from jax.experimental.pallas import tpu_sc as plsc

DIM = 64
HALF = DIM // 2
MAX_POS = 8192
THETA = 10000.0
SEQ = 8192

_w = 1.0 / (THETA ** (np.arange(0, DIM, 2, dtype=np.float64) / DIM))
_hi_ang = np.outer(np.arange(MAX_POS // 128) * 128.0, _w)
_lo_ang = np.outer(np.arange(128).astype(np.float64), _w)
_THI = np.concatenate([np.cos(_hi_ang), np.sin(_hi_ang)], axis=1).astype(np.float32)
_TLO = np.concatenate([np.cos(_lo_ang), np.sin(_lo_ang)], axis=1).astype(np.float32)

_NC, _NS = 2, 16
_NW = _NC * _NS
_CHUNK = SEQ // _NW
_G = _CHUNK // 16


@functools.partial(
    pl.kernel,
    mesh=plsc.VectorSubcoreMesh(core_axis_name="c", subcore_axis_name="s"),
    out_type=(
        jax.ShapeDtypeStruct((SEQ, DIM), jnp.float32),
        jax.ShapeDtypeStruct((SEQ, DIM), jnp.float32),
    ),
    scratch_types=[
        pltpu.VMEM((_CHUNK,), jnp.int32),
        pltpu.VMEM((MAX_POS // 128, DIM), jnp.float32),
        pltpu.VMEM((128, DIM), jnp.float32),
        pltpu.VMEM((_CHUNK, DIM), jnp.float32),
        pltpu.VMEM((_CHUNK, DIM), jnp.float32),
        pltpu.SemaphoreType.DMA,
        pltpu.SemaphoreType.DMA,
        pltpu.SemaphoreType.DMA,
    ],
    compiler_params=pltpu.CompilerParams(
        use_tc_tiling_on_sc=False, needs_layout_passes=False
    ),
)
def _rope_trig(thi_hbm, tlo_hbm, idx_hbm, cos_out, sin_out,
               idx_v, thi_v, tlo_v, cos_v, sin_v, sem_c, sem_s, sem_t):
    wid = lax.axis_index("s") * _NC + lax.axis_index("c")
    base = wid * _CHUNK
    cp1 = pltpu.async_copy(idx_hbm.at[pl.ds(base, _CHUNK)], idx_v, sem_c)
    cp2 = pltpu.async_copy(thi_hbm, thi_v, sem_s)
    cp3 = pltpu.async_copy(tlo_hbm, tlo_v, sem_t)
    cp1.wait()
    cp2.wait()
    cp3.wait()

    lanes = lax.iota(jnp.int32, 16)

    def _group(g):
        idxv = idx_v[pl.ds(g * 16, 16)]
        for j in range(16):
            p = g * 16 + j
            bvec = jnp.take_along_axis(
                idxv, jnp.full((16,), j, jnp.int32), axis=0,
                mode="promise_in_bounds",
            )
            hi = lax.shift_right_logical(bvec, 7)
            lo = lax.bitwise_and(bvec, 127)
            ch0 = plsc.load_gather(thi_v, [hi, lanes])
            ch1 = plsc.load_gather(thi_v, [hi, 16 + lanes])
            sh0 = plsc.load_gather(thi_v, [hi, 32 + lanes])
            sh1 = plsc.load_gather(thi_v, [hi, 48 + lanes])
            cl0 = plsc.load_gather(tlo_v, [lo, lanes])
            cl1 = plsc.load_gather(tlo_v, [lo, 16 + lanes])
            sl0 = plsc.load_gather(tlo_v, [lo, 32 + lanes])
            sl1 = plsc.load_gather(tlo_v, [lo, 48 + lanes])
            c0 = ch0 * cl0 - sh0 * sl0
            c1 = ch1 * cl1 - sh1 * sl1
            s0 = sh0 * cl0 + ch0 * sl0
            s1 = sh1 * cl1 + ch1 * sl1
            cos_v[p, pl.ds(0, 16)] = c0
            cos_v[p, pl.ds(16, 16)] = c1
            cos_v[p, pl.ds(32, 16)] = c0
            cos_v[p, pl.ds(48, 16)] = c1
            sin_v[p, pl.ds(0, 16)] = s0
            sin_v[p, pl.ds(16, 16)] = s1
            sin_v[p, pl.ds(32, 16)] = s0
            sin_v[p, pl.ds(48, 16)] = s1

    plsc.parallel_loop(0, _G, 1, unroll=2)(_group)

    cp_c = pltpu.async_copy(cos_v, cos_out.at[pl.ds(base, _CHUNK)], sem_c)
    cp_s = pltpu.async_copy(sin_v, sin_out.at[pl.ds(base, _CHUNK)], sem_s)
    cp_c.wait()
    cp_s.wait()


def kernel(x, position_ids):
    thi = jnp.asarray(_THI)
    tlo = jnp.asarray(_TLO)
    idx = position_ids.reshape(SEQ).astype(jnp.int32)
    cos, sin = _rope_trig(thi, tlo, idx)
    cos = cos.reshape(1, 1, SEQ, DIM).astype(x.dtype)
    sin = sin.reshape(1, 1, SEQ, DIM).astype(x.dtype)
    return cos, sin
